# Initial kernel scaffold; baseline (speedup 1.0000x reference)
import jax
import jax.numpy as jnp
from jax.experimental import pallas as pl


def kernel(achieved_goal, desired_goal, norm_mean, norm_std):
    raise NotImplementedError("write your pallas kernel here")


# speedup vs baseline: 1.4393x; 1.4393x over previous
import functools

import jax
import jax.numpy as jnp
from jax.experimental import pallas as pl

_N = 1024
_THR = 6.0


def _direction_reward(H, pfd, sel_xy, dst_xy):
    diff = dst_xy - sel_xy
    dist = jnp.sqrt(jnp.sum(diff * diff, axis=-1))
    count = jnp.sum(H * pfd[:, None], axis=0)
    wcnt = jnp.sum(H * count[None, :], axis=1)
    contrib = jnp.where(pfd > 0.5, dist / (wcnt + 1e-6), 0.0)
    unmatched = jnp.max(1.0 - pfd)
    n_groups = jnp.sum(jnp.where(count > 0.5, 1.0, 0.0)) + unmatched
    n_groups = jnp.maximum(n_groups, 1.0)
    return -(jnp.sum(contrib) + unmatched) / n_groups


def _chamfer_kernel(sv_ref, gv_ref, sxy_ref, gxy_ref, out_ref):
    sv = sv_ref[0]
    gv = gv_ref[0]
    sxy = sxy_ref[0]
    gxy = gxy_ref[0]

    xx = jnp.sum(sv * sv, axis=-1)[:, None]
    yy = jnp.sum(gv * gv, axis=-1)[None, :]
    zz = jax.lax.dot_general(sv, gv, (((1,), (1,)), ((), ())),
                             preferred_element_type=jnp.float32)
    P = xx + yy - 2.0 * zz

    iota_m = jax.lax.broadcasted_iota(jnp.int32, (_N, _N), 1)
    iota_n = jax.lax.broadcasted_iota(jnp.int32, (_N, _N), 0)
    big = jnp.int32(_N)

    min_r = jnp.min(P, axis=1)
    idx_r = jnp.min(jnp.where(P == min_r[:, None], iota_m, big), axis=1)
    H_r = (iota_m == idx_r[:, None]).astype(jnp.float32)
    pfd_r = (min_r <= _THR).astype(jnp.float32)
    sel_r = jax.lax.dot_general(H_r, gxy, (((1,), (0,)), ((), ())),
                                preferred_element_type=jnp.float32)
    r_s2g = _direction_reward(H_r, pfd_r, sel_r, sxy)

    min_c = jnp.min(P, axis=0)
    idx_c = jnp.min(jnp.where(P == min_c[None, :], iota_n, big), axis=0)
    Ht_c = (iota_m == idx_c[:, None]).astype(jnp.float32)
    pfd_c = (min_c <= _THR).astype(jnp.float32)
    sel_c = jax.lax.dot_general(Ht_c, sxy, (((1,), (0,)), ((), ())),
                                preferred_element_type=jnp.float32)
    r_g2s = _direction_reward(Ht_c, pfd_c, sel_c, gxy)

    out_ref[0, 0, :] = jnp.full((128,), (r_g2s + r_s2g) * 0.5, jnp.float32)


@jax.jit
def kernel(achieved_goal, desired_goal, norm_mean, norm_std):
    state = achieved_goal * norm_std + norm_mean
    goal = desired_goal * norm_std + norm_mean
    bs, n_views, n_particles, _ = state.shape
    bv = bs * n_views

    sv = state[..., 5:9].reshape(bv, n_particles, 4)
    gv = goal[..., 5:9].reshape(bv, n_particles, 4)
    sxy = state[..., :2].reshape(bv, n_particles, 2)
    gxy = goal[..., :2].reshape(bv, n_particles, 2)

    out = pl.pallas_call(
        _chamfer_kernel,
        grid=(bv,),
        in_specs=[
            pl.BlockSpec((1, n_particles, 4), lambda i: (i, 0, 0)),
            pl.BlockSpec((1, n_particles, 4), lambda i: (i, 0, 0)),
            pl.BlockSpec((1, n_particles, 2), lambda i: (i, 0, 0)),
            pl.BlockSpec((1, n_particles, 2), lambda i: (i, 0, 0)),
        ],
        out_specs=pl.BlockSpec((1, 1, 128), lambda i: (i, 0, 0)),
        out_shape=jax.ShapeDtypeStruct((bv, 1, 128), jnp.float32),
    )(sv, gv, sxy, gxy)

    reward = out[:, 0, 0].reshape(bs, n_views).mean(axis=1)
    return reward[:, None]
